# Initial kernel scaffold; baseline (speedup 1.0000x reference)
import jax
import jax.numpy as jnp
from jax.experimental import pallas as pl


def kernel(x, W, A, B, mag):
    raise NotImplementedError("write your pallas kernel here")


# speedup vs baseline: 9.2390x; 9.2390x over previous
import functools

import jax
import jax.numpy as jnp
from jax import lax
from jax.experimental import pallas as pl
from jax.experimental.pallas import tpu as pltpu
from jax.experimental.pallas import tpu_sc as plsc

_VOCAB = 1000000
_D = 64
_R = 16
_SCALING = 1.0

_TILE = 8192
_NSTEP = (_VOCAB + _TILE - 1) // _TILE


def _ss_body(w_ref, a_ref, bt_ref, ss_ref):
    i = pl.program_id(0)
    delta = lax.dot_general(
        a_ref[...], bt_ref[...], (((0,), (0,)), ((), ())),
        preferred_element_type=jnp.float32)
    d = w_ref[...] + delta * _SCALING
    v = i * _TILE + lax.broadcasted_iota(jnp.int32, (_TILE, _D), 0)
    d2 = jnp.where(v < _VOCAB, d * d, 0.0)
    part = jnp.broadcast_to(jnp.sum(d2, axis=0, keepdims=True), (8, _D))

    @pl.when(i == 0)
    def _():
        ss_ref[...] = part

    @pl.when(i > 0)
    def _():
        ss_ref[...] += part


def _table_body(w_ref, a_ref, bt_ref, ss_ref, mag_ref, out_ref):
    delta = lax.dot_general(
        a_ref[...], bt_ref[...], (((0,), (0,)), ((), ())),
        preferred_element_type=jnp.float32)
    d = w_ref[...] + delta * _SCALING
    scale = mag_ref[...] * jnp.sqrt(ss_ref[0:1, :])
    out_ref[...] = d * scale


_NW = 32
_NTOK = 16384 * 50
_PER_W = _NTOK // _NW
_SUB = 128
_GPC = 8
_CHUNK = _SUB * _GPC
_NCH = _PER_W // _CHUNK


def _gather_body(table_hbm, idx_hbm, out_hbm, idx_v, rows_v, sem):
    wid = lax.axis_index("s") * 2 + lax.axis_index("c")
    base_row = wid * (_PER_W // _SUB)

    def chunk(ci, carry):
        row = base_row + ci * _GPC
        pltpu.sync_copy(idx_hbm.at[pl.ds(row, _GPC)], idx_v)
        copies = [
            pltpu.async_copy(table_hbm.at[idx_v.at[j]], rows_v.at[j], sem)
            for j in range(_GPC)
        ]
        for c in copies:
            c.wait()
        pltpu.sync_copy(rows_v, out_hbm.at[pl.ds(row, _GPC)])
        return carry

    lax.fori_loop(0, _NCH, chunk, 0)


@functools.cache
def _make_gather():
    mesh = plsc.VectorSubcoreMesh(core_axis_name="c", subcore_axis_name="s")
    return functools.partial(
        pl.kernel,
        out_type=jax.ShapeDtypeStruct((_NTOK // _SUB, _SUB, _D), jnp.float32),
        mesh=mesh,
        scratch_types=[
            pltpu.VMEM((_GPC, _SUB), jnp.int32),
            pltpu.VMEM((_GPC, _SUB, _D), jnp.float32),
            pltpu.SemaphoreType.DMA,
        ],
        compiler_params=pltpu.CompilerParams(use_tc_tiling_on_sc=False),
    )(_gather_body)


def kernel(x, W, A, B, mag):
    Bt = B.T
    mag2 = mag.reshape(1, _D)
    ss = pl.pallas_call(
        _ss_body,
        grid=(_NSTEP,),
        in_specs=[
            pl.BlockSpec((_TILE, _D), lambda i: (i, 0)),
            pl.BlockSpec((_R, _TILE), lambda i: (0, i)),
            pl.BlockSpec((_R, _D), lambda i: (0, 0)),
        ],
        out_specs=pl.BlockSpec((8, _D), lambda i: (0, 0)),
        out_shape=jax.ShapeDtypeStruct((8, _D), jnp.float32),
    )(W, A, Bt)

    table = pl.pallas_call(
        _table_body,
        grid=(_NSTEP,),
        in_specs=[
            pl.BlockSpec((_TILE, _D), lambda i: (i, 0)),
            pl.BlockSpec((_R, _TILE), lambda i: (0, i)),
            pl.BlockSpec((_R, _D), lambda i: (0, 0)),
            pl.BlockSpec((8, _D), lambda i: (0, 0)),
            pl.BlockSpec((1, _D), lambda i: (0, 0)),
        ],
        out_specs=pl.BlockSpec((_TILE, _D), lambda i: (i, 0)),
        out_shape=jax.ShapeDtypeStruct((_VOCAB, _D), jnp.float32),
    )(W, A, Bt, ss, mag2)

    idx = x.astype(jnp.int32).reshape(_NTOK // _SUB, _SUB)
    out = _make_gather()(table, idx)
    return out.reshape(16384, 50, _D)


# speedup vs baseline: 17.6990x; 1.9157x over previous
import functools

import jax
import jax.numpy as jnp
from jax import lax
from jax.experimental import pallas as pl
from jax.experimental.pallas import tpu as pltpu
from jax.experimental.pallas import tpu_sc as plsc

_VOCAB = 1000000
_D = 64
_R = 16
_SCALING = 1.0

_TILE = 8192
_NSTEP = (_VOCAB + _TILE - 1) // _TILE
_NPAIR = 62
_SPLIT = _NPAIR * _TILE
_BOFF = (_NSTEP - _NPAIR) * _TILE


def _ss_body(wt_ref, a_ref, b_ref, ss_ref):
    i = pl.program_id(0)
    dt = wt_ref[...] + lax.dot_general(
        b_ref[...], a_ref[...], (((1,), (0,)), ((), ())),
        preferred_element_type=jnp.float32) * _SCALING
    v = i * _TILE + lax.broadcasted_iota(jnp.int32, (_D, _TILE), 1)
    d2 = jnp.where(v < _VOCAB, dt * dt, 0.0)
    part = jnp.broadcast_to(jnp.sum(d2, axis=1, keepdims=True), (_D, 8))

    @pl.when(i == 0)
    def _():
        ss_ref[...] = part

    @pl.when(i > 0)
    def _():
        ss_ref[...] += part


def _table_body(wta_ref, wtb_ref, aa_ref, ab_ref, b_ref, ss_ref, mag_ref,
                out_ref):
    scale = mag_ref[...] * jnp.sqrt(ss_ref[:, 0:1])
    dta = wta_ref[...] + lax.dot_general(
        b_ref[...], aa_ref[...], (((1,), (0,)), ((), ())),
        preferred_element_type=jnp.float32) * _SCALING
    dtb = wtb_ref[...] + lax.dot_general(
        b_ref[...], ab_ref[...], (((1,), (0,)), ((), ())),
        preferred_element_type=jnp.float32) * _SCALING
    packed = jnp.concatenate([dta * scale, dtb * scale], axis=0)
    out_ref[...] = packed.T


_NW = 32
_NTOK = 16384 * 50
_PER_W = _NTOK // _NW
_SUB = 128
_GPC = 8
_CHUNK = _SUB * _GPC
_NCH = _PER_W // _CHUNK


def _gather_body(table_hbm, idx_hbm, out_hbm, idx_v, rows_v, sem):
    wid = lax.axis_index("s") * 2 + lax.axis_index("c")
    base_row = wid * (_PER_W // _SUB)

    def chunk(ci, carry):
        row = base_row + ci * _GPC
        pltpu.sync_copy(idx_hbm.at[pl.ds(row, _GPC)], idx_v)
        for j in range(_GPC):
            for k in range(_SUB // 16):
                v = idx_v[j, pl.ds(k * 16, 16)]
                u = v + v - jnp.where(v >= _SPLIT, 2 * _BOFF - 1, 0)
                idx_v[j, pl.ds(k * 16, 16)] = u
        copies = [
            pltpu.async_copy(table_hbm.at[idx_v.at[j]], rows_v.at[j], sem)
            for j in range(_GPC)
        ]
        for c in copies:
            c.wait()
        pltpu.sync_copy(rows_v, out_hbm.at[pl.ds(row, _GPC)])
        return carry

    lax.fori_loop(0, _NCH, chunk, 0)


@functools.cache
def _make_gather():
    mesh = plsc.VectorSubcoreMesh(core_axis_name="c", subcore_axis_name="s")
    return functools.partial(
        pl.kernel,
        out_type=jax.ShapeDtypeStruct((_NTOK // _SUB, _SUB, _D), jnp.float32),
        mesh=mesh,
        scratch_types=[
            pltpu.VMEM((_GPC, _SUB), jnp.int32),
            pltpu.VMEM((_GPC, _SUB, _D), jnp.float32),
            pltpu.SemaphoreType.DMA,
        ],
        compiler_params=pltpu.CompilerParams(use_tc_tiling_on_sc=False),
    )(_gather_body)


def kernel(x, W, A, B, mag):
    Wt = W.T
    mag_col = mag.reshape(_D, 1)
    ss = pl.pallas_call(
        _ss_body,
        grid=(_NSTEP,),
        in_specs=[
            pl.BlockSpec((_D, _TILE), lambda i: (0, i)),
            pl.BlockSpec((_R, _TILE), lambda i: (0, i)),
            pl.BlockSpec((_D, _R), lambda i: (0, 0)),
        ],
        out_specs=pl.BlockSpec((_D, 8), lambda i: (0, 0)),
        out_shape=jax.ShapeDtypeStruct((_D, 8), jnp.float32),
    )(Wt, A, B)

    table = pl.pallas_call(
        _table_body,
        grid=(_NPAIR,),
        in_specs=[
            pl.BlockSpec((_D, _TILE), lambda i: (0, i)),
            pl.BlockSpec((_D, _TILE), lambda i: (0, i + _NSTEP - _NPAIR)),
            pl.BlockSpec((_R, _TILE), lambda i: (0, i)),
            pl.BlockSpec((_R, _TILE), lambda i: (0, i + _NSTEP - _NPAIR)),
            pl.BlockSpec((_D, _R), lambda i: (0, 0)),
            pl.BlockSpec((_D, 8), lambda i: (0, 0)),
            pl.BlockSpec((_D, 1), lambda i: (0, 0)),
        ],
        out_specs=pl.BlockSpec((_TILE, 2 * _D), lambda i: (i, 0)),
        out_shape=jax.ShapeDtypeStruct((_SPLIT, 2 * _D), jnp.float32),
    )(Wt, Wt, A, A, B, ss, mag_col)

    table64 = table.reshape(2 * _SPLIT, _D)
    idx = x.astype(jnp.int32).reshape(_NTOK // _SUB, _SUB)
    out = _make_gather()(table64, idx)
    return out.reshape(16384, 50, _D)


# speedup vs baseline: 24.9255x; 1.4083x over previous
import functools

import jax
import jax.numpy as jnp
from jax import lax
from jax.experimental import pallas as pl
from jax.experimental.pallas import tpu as pltpu
from jax.experimental.pallas import tpu_sc as plsc

_VOCAB = 1000000
_D = 64
_R = 16
_SCALING = 1.0

_TILE = 8192
_NSTEP = (_VOCAB + _TILE - 1) // _TILE
_NPAIR = 62
_SPLIT = _NPAIR * _TILE
_BOFF = (_NSTEP - _NPAIR) * _TILE

_B_ROWS = 16384
_SEQ = 50
_LD = _SEQ * _D
_TBLK = 256
_NTB = _B_ROWS // _TBLK


def _table_body(wta_ref, wtb_ref, aa_ref, ab_ref, b_ref, out_ref, ss_ref):
    i = pl.program_id(0)
    dta = wta_ref[...] + lax.dot_general(
        b_ref[...], aa_ref[...], (((1,), (0,)), ((), ())),
        preferred_element_type=jnp.float32) * _SCALING
    dtb = wtb_ref[...] + lax.dot_general(
        b_ref[...], ab_ref[...], (((1,), (0,)), ((), ())),
        preferred_element_type=jnp.float32) * _SCALING
    packed = jnp.concatenate([dta, dtb], axis=0)
    out_ref[...] = packed.T
    vb = (_NSTEP - _NPAIR + i) * _TILE + lax.broadcasted_iota(
        jnp.int32, (_D, _TILE), 1)
    d2 = dta * dta + jnp.where(
        (vb >= _SPLIT) & (vb < _VOCAB), dtb * dtb, 0.0)
    part = jnp.broadcast_to(jnp.sum(d2, axis=1, keepdims=True), (_D, 8))

    @pl.when(i == 0)
    def _():
        ss_ref[...] = part

    @pl.when(i > 0)
    def _():
        ss_ref[...] += part


_NW = 32
_NTOK = _B_ROWS * _SEQ
_PER_W = _NTOK // _NW
_SUB = 128
_GPC = 8
_CHUNK = _SUB * _GPC
_NCH = _PER_W // _CHUNK


def _gather_body(table_hbm, idx_hbm, out_hbm, idx_v, rows_v, sem):
    wid = lax.axis_index("s") * 2 + lax.axis_index("c")
    base_row = wid * (_PER_W // _SUB)

    def chunk(ci, carry):
        row = base_row + ci * _GPC
        pltpu.sync_copy(idx_hbm.at[pl.ds(row, _GPC)], idx_v)
        for j in range(_GPC):
            for k in range(_SUB // 16):
                v = idx_v[j, pl.ds(k * 16, 16)]
                u = v + v - jnp.where(v >= _SPLIT, 2 * _BOFF - 1, 0)
                idx_v[j, pl.ds(k * 16, 16)] = u
        copies = [
            pltpu.async_copy(table_hbm.at[idx_v.at[j]], rows_v.at[j], sem)
            for j in range(_GPC)
        ]
        for c in copies:
            c.wait()
        pltpu.sync_copy(rows_v, out_hbm.at[pl.ds(row, _GPC)])
        return carry

    lax.fori_loop(0, _NCH, chunk, 0)


@functools.cache
def _make_gather():
    mesh = plsc.VectorSubcoreMesh(core_axis_name="c", subcore_axis_name="s")
    return functools.partial(
        pl.kernel,
        out_type=jax.ShapeDtypeStruct((_NTOK // _SUB, _SUB, _D), jnp.float32),
        mesh=mesh,
        scratch_types=[
            pltpu.VMEM((_GPC, _SUB), jnp.int32),
            pltpu.VMEM((_GPC, _SUB, _D), jnp.float32),
            pltpu.SemaphoreType.DMA,
        ],
        compiler_params=pltpu.CompilerParams(use_tc_tiling_on_sc=False),
    )(_gather_body)


def _trans_body(x_ref, ss_ref, mag_ref, y_ref):
    scale_col = mag_ref[...] * jnp.sqrt(ss_ref[:, 0:1])
    sc = jnp.concatenate([scale_col] * _SEQ, axis=0)
    y_ref[...] = x_ref[...].T * sc


def kernel(x, W, A, B, mag):
    Wt = W.T
    mag_col = mag.reshape(_D, 1)
    table, ss = pl.pallas_call(
        _table_body,
        grid=(_NPAIR,),
        in_specs=[
            pl.BlockSpec((_D, _TILE), lambda i: (0, i)),
            pl.BlockSpec((_D, _TILE), lambda i: (0, i + _NSTEP - _NPAIR)),
            pl.BlockSpec((_R, _TILE), lambda i: (0, i)),
            pl.BlockSpec((_R, _TILE), lambda i: (0, i + _NSTEP - _NPAIR)),
            pl.BlockSpec((_D, _R), lambda i: (0, 0)),
        ],
        out_specs=[
            pl.BlockSpec((_TILE, 2 * _D), lambda i: (i, 0)),
            pl.BlockSpec((_D, 8), lambda i: (0, 0)),
        ],
        out_shape=[
            jax.ShapeDtypeStruct((_SPLIT, 2 * _D), jnp.float32),
            jax.ShapeDtypeStruct((_D, 8), jnp.float32),
        ],
    )(Wt, Wt, A, A, B)

    table64 = table.reshape(2 * _SPLIT, _D)
    idx = x.astype(jnp.int32).reshape(_NTOK // _SUB, _SUB)
    rows = _make_gather()(table64, idx)

    xv = rows.reshape(_B_ROWS, _LD)
    y = pl.pallas_call(
        _trans_body,
        grid=(_NTB,),
        in_specs=[
            pl.BlockSpec((_TBLK, _LD), lambda i: (i, 0)),
            pl.BlockSpec((_D, 8), lambda i: (0, 0)),
            pl.BlockSpec((_D, 1), lambda i: (0, 0)),
        ],
        out_specs=pl.BlockSpec((_LD, _TBLK), lambda i: (0, i)),
        out_shape=jax.ShapeDtypeStruct((_LD, _B_ROWS), jnp.float32),
    )(xv, ss, mag_col)

    return jnp.transpose(y.reshape(_SEQ, _D, _B_ROWS), (2, 0, 1))


# speedup vs baseline: 30.8650x; 1.2383x over previous
import functools

import jax
import jax.numpy as jnp
from jax import lax
from jax.experimental import pallas as pl
from jax.experimental.pallas import tpu as pltpu
from jax.experimental.pallas import tpu_sc as plsc

_VOCAB = 1000000
_D = 64
_R = 16
_SCALING = 1.0

_TILE = 8192
_NSTEP = (_VOCAB + _TILE - 1) // _TILE
_NPAIR = 62
_SPLIT = _NPAIR * _TILE
_BOFF = (_NSTEP - _NPAIR) * _TILE

_B_ROWS = 16384
_SEQ = 50
_LD = _SEQ * _D
_TBLK = 256
_NTB = _B_ROWS // _TBLK


def _table_body(wta_ref, wtb_ref, aa_ref, ab_ref, b_ref, out_ref, ss_ref):
    i = pl.program_id(0)
    dta = wta_ref[...] + lax.dot_general(
        b_ref[...], aa_ref[...], (((1,), (0,)), ((), ())),
        preferred_element_type=jnp.float32) * _SCALING
    dtb = wtb_ref[...] + lax.dot_general(
        b_ref[...], ab_ref[...], (((1,), (0,)), ((), ())),
        preferred_element_type=jnp.float32) * _SCALING
    packed = jnp.concatenate([dta, dtb], axis=0)
    out_ref[...] = packed.T
    vb = (_NSTEP - _NPAIR + i) * _TILE + lax.broadcasted_iota(
        jnp.int32, (_D, _TILE), 1)
    d2 = dta * dta + jnp.where(
        (vb >= _SPLIT) & (vb < _VOCAB), dtb * dtb, 0.0)
    part = jnp.broadcast_to(jnp.sum(d2, axis=1, keepdims=True), (_D, 8))

    @pl.when(i == 0)
    def _():
        ss_ref[...] = part

    @pl.when(i > 0)
    def _():
        ss_ref[...] += part


_NW = 32
_NTOK = _B_ROWS * _SEQ
_PER_W = _NTOK // _NW
_SUB = 128
_GPC = 8
_CHUNK = _SUB * _GPC
_NCH = _PER_W // _CHUNK


def _gather_body(table_hbm, idx_hbm, out_hbm, idx_v, rows_v, sem):
    wid = lax.axis_index("s") * 2 + lax.axis_index("c")
    base_row = wid * (_PER_W // _SUB)

    def chunk(ci, carry):
        row = base_row + ci * _GPC
        pltpu.sync_copy(idx_hbm.at[pl.ds(row, _GPC)], idx_v)
        for j in range(_GPC):
            for k in range(_SUB // 16):
                v = idx_v[j, pl.ds(k * 16, 16)]
                u = v + v - jnp.where(v >= _SPLIT, 2 * _BOFF - 1, 0)
                idx_v[j, pl.ds(k * 16, 16)] = u
        copies = [
            pltpu.async_copy(table_hbm.at[idx_v.at[j]], rows_v.at[j], sem)
            for j in range(_GPC)
        ]
        for c in copies:
            c.wait()
        pltpu.sync_copy(rows_v, out_hbm.at[pl.ds(row, _GPC)])
        return carry

    lax.fori_loop(0, _NCH, chunk, 0)


@functools.cache
def _make_gather():
    mesh = plsc.VectorSubcoreMesh(core_axis_name="c", subcore_axis_name="s")
    return functools.partial(
        pl.kernel,
        out_type=jax.ShapeDtypeStruct((_NTOK // _SUB, _SUB, _D), jnp.float32),
        mesh=mesh,
        scratch_types=[
            pltpu.VMEM((_GPC, _SUB), jnp.int32),
            pltpu.VMEM((_GPC, _SUB, _D), jnp.float32),
            pltpu.SemaphoreType.DMA,
        ],
        compiler_params=pltpu.CompilerParams(use_tc_tiling_on_sc=False),
    )(_gather_body)


def _trans_body(x_ref, ss_ref, mag_ref, y_ref):
    scale_col = mag_ref[...] * jnp.sqrt(ss_ref[:, 0:1])
    sc = jnp.concatenate([scale_col] * _SEQ, axis=0)
    x3 = x_ref[...].reshape(_TBLK, _SEQ // 2, 128)
    parts = [x3[:, g, :].T for g in range(_SEQ // 2)]
    y_ref[...] = jnp.concatenate(parts, axis=0) * sc


def kernel(x, W, A, B, mag):
    Wt = W.T
    mag_col = mag.reshape(_D, 1)
    table, ss = pl.pallas_call(
        _table_body,
        grid=(_NPAIR,),
        in_specs=[
            pl.BlockSpec((_D, _TILE), lambda i: (0, i)),
            pl.BlockSpec((_D, _TILE), lambda i: (0, i + _NSTEP - _NPAIR)),
            pl.BlockSpec((_R, _TILE), lambda i: (0, i)),
            pl.BlockSpec((_R, _TILE), lambda i: (0, i + _NSTEP - _NPAIR)),
            pl.BlockSpec((_D, _R), lambda i: (0, 0)),
        ],
        out_specs=[
            pl.BlockSpec((_TILE, 2 * _D), lambda i: (i, 0)),
            pl.BlockSpec((_D, 8), lambda i: (0, 0)),
        ],
        out_shape=[
            jax.ShapeDtypeStruct((_SPLIT, 2 * _D), jnp.float32),
            jax.ShapeDtypeStruct((_D, 8), jnp.float32),
        ],
    )(Wt, Wt, A, A, B)

    table64 = table.reshape(2 * _SPLIT, _D)
    idx = x.astype(jnp.int32).reshape(_NTOK // _SUB, _SUB)
    rows = _make_gather()(table64, idx)

    xv = rows.reshape(_NTOK // 2, 128)
    y = pl.pallas_call(
        _trans_body,
        grid=(_NTB,),
        in_specs=[
            pl.BlockSpec((_TBLK * _SEQ // 2, 128), lambda i: (i, 0)),
            pl.BlockSpec((_D, 8), lambda i: (0, 0)),
            pl.BlockSpec((_D, 1), lambda i: (0, 0)),
        ],
        out_specs=pl.BlockSpec((_LD, _TBLK), lambda i: (0, i)),
        out_shape=jax.ShapeDtypeStruct((_LD, _B_ROWS), jnp.float32),
    )(xv, ss, mag_col)

    return jnp.transpose(y.reshape(_SEQ, _D, _B_ROWS), (2, 0, 1))
